# baseline (speedup 1.0000x reference)
import functools

import jax
import jax.numpy as jnp
from jax import lax
from jax.experimental import pallas as pl
from jax.experimental.pallas import tpu as pltpu
from jax.experimental.pallas import tpu_sc as plsc

V = 1000
D = 128
B = 1024
L = 20
VP = 1024
KC = VP // 128
N = B * L

_INFO = plsc.get_sparse_core_info()
_NC = _INFO.num_cores
_NS = _INFO.num_subcores
LN = _INFO.num_lanes
NW = _NC * _NS
BW = B // NW
EW = BW * L
RW = EW // 128


def _ht_body(emb_ref, w_ref, ht_ref):
    zpad = jnp.zeros((VP - V, D), jnp.float32)
    wp = jnp.concatenate([w_ref[...], zpad], axis=0)
    ep = jnp.concatenate([emb_ref[...], zpad], axis=0)
    a = lax.dot_general(wp, ep, (((1,), (1,)), ((), ())),
                        preferred_element_type=jnp.float32)
    row_c = lax.broadcasted_iota(jnp.int32, (VP, VP), 0)
    a_msk = jnp.where(row_c < V, a, -1e30)
    m = jnp.max(a_msk, axis=0, keepdims=True)
    lz = m + jnp.log(jnp.sum(jnp.exp(a_msk - m), axis=0, keepdims=True))
    h = lz - a
    for k in range(KC):
        ht_ref[k, :, :] = h[:, 128 * k:128 * (k + 1)]


_ht_call = pl.pallas_call(
    _ht_body,
    out_shape=jax.ShapeDtypeStruct((KC, VP, 128), jnp.float32),
)


_mesh = plsc.VectorSubcoreMesh(core_axis_name="c", subcore_axis_name="s")


@functools.partial(
    pl.kernel,
    mesh=_mesh,
    compiler_params=pltpu.CompilerParams(use_tc_tiling_on_sc=False,
                                         needs_layout_passes=False),
    out_type=jax.ShapeDtypeStruct((NW * LN,), jnp.float32),
    scratch_types=[
        pltpu.VMEM((EW,), jnp.int32),
        pltpu.VMEM((BW,), jnp.int32),
        pltpu.VMEM((RW, 128), jnp.float32),
        pltpu.VMEM((LN,), jnp.float32),
        pltpu.SemaphoreType.DMA,
    ],
)
def _sc_gather(htf_hbm, ctx_hbm, ctr_hbm, out_hbm, idx_v, ctr_v, hv, acc_v, sem):
    wid = lax.axis_index("s") * _NC + lax.axis_index("c")
    pltpu.sync_copy(ctx_hbm.at[pl.ds(wid * EW, EW)], idx_v)
    pltpu.sync_copy(ctr_hbm.at[pl.ds(wid * BW, BW)], ctr_v)

    lane = lax.iota(jnp.int32, LN)

    def fbody(j, _):
        p = j * LN + lane
        c = plsc.load_gather(ctr_v, [p // L])
        i = idx_v[pl.ds(j * LN, LN)]
        idx_v[pl.ds(j * LN, LN)] = (i >> 7) * (VP * 128) + (i & 127) + c * 128
        return 0

    lax.fori_loop(0, EW // LN, fbody, 0)

    cps = [pltpu.async_copy(htf_hbm.at[idx_v.at[pl.ds(r * 128, 128)]],
                            hv.at[r], sem)
           for r in range(RW)]
    for cp in cps:
        cp.wait()
    acc = jnp.zeros((LN,), jnp.float32)
    for r in range(RW):
        for c in range(128 // LN):
            acc = acc + hv[r, pl.ds(c * LN, LN)]
    acc_v[...] = acc
    pltpu.sync_copy(acc_v, out_hbm.at[pl.ds(wid * LN, LN)])


def _fin_body(p_ref, o_ref):
    o_ref[...] = jnp.sum(p_ref[...]).reshape(1, 1) * (1.0 / N)


_fin_call = pl.pallas_call(
    _fin_body,
    out_shape=jax.ShapeDtypeStruct((1, 1), jnp.float32),
)


def kernel(center, context, emb_in, W_out):
    ht = _ht_call(emb_in, W_out)
    parts = _sc_gather(ht.reshape(KC * VP * 128), context.reshape(N), center)
    return _fin_call(parts)[0, 0]


# speedup vs baseline: 1.0009x; 1.0009x over previous
import functools

import jax
import jax.numpy as jnp
from jax import lax
from jax.experimental import pallas as pl
from jax.experimental.pallas import tpu as pltpu
from jax.experimental.pallas import tpu_sc as plsc

V = 1000
D = 128
B = 1024
L = 20
VP = 1024
KC = VP // 128
N = B * L

_INFO = plsc.get_sparse_core_info()
_NC = _INFO.num_cores
_NS = _INFO.num_subcores
LN = _INFO.num_lanes
NW = _NC * _NS
BW = B // NW
EW = BW * L
RW = EW // 128


def _ht_body(emb_ref, w_ref, ht_ref):
    zpad = jnp.zeros((VP - V, D), jnp.float32)
    wp = jnp.concatenate([w_ref[...], zpad], axis=0)
    ep = jnp.concatenate([emb_ref[...], zpad], axis=0)
    a = lax.dot_general(wp, ep, (((1,), (1,)), ((), ())),
                        preferred_element_type=jnp.float32)
    row_c = lax.broadcasted_iota(jnp.int32, (VP, VP), 0)
    a_msk = jnp.where(row_c < V, a, -1e30)
    m = jnp.max(a_msk, axis=0, keepdims=True)
    lz = m + jnp.log(jnp.sum(jnp.exp(a_msk - m), axis=0, keepdims=True))
    h = lz - a
    for k in range(KC):
        ht_ref[k, :, :] = h[:, 128 * k:128 * (k + 1)]


_ht_call = pl.pallas_call(
    _ht_body,
    out_shape=jax.ShapeDtypeStruct((KC, VP, 128), jnp.float32),
)


_mesh = plsc.VectorSubcoreMesh(core_axis_name="c", subcore_axis_name="s")


@functools.partial(
    pl.kernel,
    mesh=_mesh,
    compiler_params=pltpu.CompilerParams(use_tc_tiling_on_sc=False,
                                         needs_layout_passes=False),
    out_type=jax.ShapeDtypeStruct((NW * LN,), jnp.float32),
    scratch_types=[
        pltpu.VMEM((EW,), jnp.int32),
        pltpu.VMEM((RW, 128), jnp.float32),
        pltpu.VMEM((LN,), jnp.float32),
        pltpu.SemaphoreType.DMA,
    ],
)
def _sc_gather(htf_hbm, fidx_hbm, out_hbm, idx_v, hv, acc_v, sem):
    wid = lax.axis_index("s") * _NC + lax.axis_index("c")
    pltpu.sync_copy(fidx_hbm.at[pl.ds(wid * EW, EW)], idx_v)

    cps = [pltpu.async_copy(htf_hbm.at[idx_v.at[pl.ds(r * 128, 128)]],
                            hv.at[r], sem)
           for r in range(RW)]
    for cp in cps:
        cp.wait()
    acc = jnp.zeros((LN,), jnp.float32)
    for r in range(RW):
        for c in range(128 // LN):
            acc = acc + hv[r, pl.ds(c * LN, LN)]
    acc_v[...] = acc
    pltpu.sync_copy(acc_v, out_hbm.at[pl.ds(wid * LN, LN)])


def _fin_body(p_ref, o_ref):
    o_ref[...] = jnp.sum(p_ref[...]).reshape(1, 1) * (1.0 / N)


_fin_call = pl.pallas_call(
    _fin_body,
    out_shape=jax.ShapeDtypeStruct((1, 1), jnp.float32),
)


def kernel(center, context, emb_in, W_out):
    ht = _ht_call(emb_in, W_out)
    fidx = ((context >> 7) * (VP * 128) + (context & 127)
            + center[:, None] * 128).reshape(N)
    parts = _sc_gather(ht.reshape(KC * VP * 128), fidx)
    return _fin_call(parts)[0, 0]
